# baseline (speedup 1.0000x reference)
import jax
import jax.numpy as jnp
from jax import lax
from jax.experimental import pallas as pl
from jax.experimental.pallas import tpu as pltpu
from jax.experimental.pallas import tpu_sc as plsc

_NUM_HEADS = 32
_HEAD_SIZE = 128
_NUM_KV_HEADS = 8
_NUM_REQS = 8
_KV_DIM = _NUM_KV_HEADS * _HEAD_SIZE
_LANES = 16


def _scatter_body(hs_hbm, qsl_hbm, sm_hbm, cache_ref,
                  qsl_v, sm_v, last_v, slots_v, idx_src_v, idx_dst_v,
                  g3, sem):
    num_slots = cache_ref.shape[0] // 2
    c = lax.axis_index("c")
    s = lax.axis_index("s")

    @pl.when(jnp.logical_and(c == 0, s == 0))
    def _():
        pltpu.sync_copy(qsl_hbm, qsl_v)
        pltpu.sync_copy(sm_hbm, sm_v)
        lanes = lax.iota(jnp.int32, _LANES)
        req = lax.bitwise_and(lanes, _NUM_REQS - 1)
        ends = plsc.load_gather(qsl_v, [req + 1])
        last = ends - 1
        slots = plsc.load_gather(sm_v, [last])
        last_v[...] = last
        slots_v[...] = slots
        win = req
        for j in range(_NUM_REQS):
            win = jnp.where(slots == slots[j], j, win)
        last_win = plsc.load_gather(last_v, [win])
        idx_src_v[...] = last_win
        pltpu.async_copy(hs_hbm.at[idx_src_v], g3.reshape(_LANES, 2 * _KV_DIM),
                         sem).wait()
        sp = plsc.load_gather(slots_v, [lax.shift_right_logical(lanes, 1)])
        idx_dst_v[...] = sp + num_slots * lax.bitwise_and(lanes, 1)
        pltpu.async_copy(g3.at[pl.ds(0, _LANES)], cache_ref.at[idx_dst_v],
                         sem).wait()


def _make_scatter(total_tokens):
    mesh = plsc.VectorSubcoreMesh(core_axis_name="c", subcore_axis_name="s")
    return pl.kernel(
        _scatter_body,
        out_type=(),
        mesh=mesh,
        compiler_params=pltpu.CompilerParams(
            needs_layout_passes=False,
            use_tc_tiling_on_sc=True,
        ),
        scratch_types=[
            pltpu.VMEM((_LANES,), jnp.int32),
            pltpu.VMEM((total_tokens,), jnp.int32),
            pltpu.VMEM((_LANES,), jnp.int32),
            pltpu.VMEM((_LANES,), jnp.int32),
            pltpu.VMEM((_LANES,), jnp.int32),
            pltpu.VMEM((_LANES,), jnp.int32),
            pltpu.VMEM((2 * _LANES, _NUM_KV_HEADS, _HEAD_SIZE), jnp.float32),
            pltpu.SemaphoreType.DMA,
        ],
    )


def _copy_block(in_ref, out_ref):
    out_ref[...] = in_ref[...]


_TC_BLK = 512


def _tc_copy(cache3):
    total_rows = cache3.shape[0]
    return pl.pallas_call(
        _copy_block,
        grid=(total_rows // _TC_BLK,),
        in_specs=[pl.BlockSpec((_TC_BLK, _NUM_KV_HEADS, _HEAD_SIZE),
                               lambda i: (i, 0, 0))],
        out_specs=pl.BlockSpec((_TC_BLK, _NUM_KV_HEADS, _HEAD_SIZE),
                               lambda i: (i, 0, 0)),
        out_shape=jax.ShapeDtypeStruct(cache3.shape, cache3.dtype),
        compiler_params=pltpu.CompilerParams(
            dimension_semantics=("arbitrary",),
        ),
    )(cache3)


def kernel(hidden_states, kv_cache, query_start_loc, slot_mapping):
    total_tokens = hidden_states.shape[0]
    num_slots = kv_cache.shape[1]
    qslp = jnp.pad(query_start_loc, (0, _LANES - query_start_loc.shape[0]))
    cache3 = kv_cache.reshape(2 * num_slots, _NUM_KV_HEADS, _HEAD_SIZE)
    cache_ref = jax.new_ref(_tc_copy(cache3))
    _make_scatter(total_tokens)(hidden_states, qslp, slot_mapping, cache_ref)
    new_cache = jax.freeze(cache_ref).reshape(kv_cache.shape)
    output = jnp.zeros((_NUM_REQS, _NUM_HEADS * _HEAD_SIZE),
                       dtype=hidden_states.dtype)
    return new_cache, output


# speedup vs baseline: 1.0941x; 1.0941x over previous
import jax
import jax.numpy as jnp
from jax import lax
from jax.experimental import pallas as pl
from jax.experimental.pallas import tpu as pltpu
from jax.experimental.pallas import tpu_sc as plsc

_NUM_HEADS = 32
_HEAD_SIZE = 128
_NUM_KV_HEADS = 8
_NUM_REQS = 8
_KV_DIM = _NUM_KV_HEADS * _HEAD_SIZE
_LANES = 16


def _scatter_body(hs_hbm, qsl_hbm, sm_hbm, cache_ref,
                  qsl_v, sm_v, last_v, slots_v, idx_src_v, idx_dst_v,
                  g3, sem):
    num_slots = cache_ref.shape[0] // 2
    c = lax.axis_index("c")
    s = lax.axis_index("s")

    @pl.when(jnp.logical_and(c == 0, s == 0))
    def _():
        pltpu.sync_copy(qsl_hbm, qsl_v)
        pltpu.sync_copy(sm_hbm, sm_v)
        lanes = lax.iota(jnp.int32, _LANES)
        req = lax.bitwise_and(lanes, _NUM_REQS - 1)
        ends = plsc.load_gather(qsl_v, [req + 1])
        last = ends - 1
        slots = plsc.load_gather(sm_v, [last])
        last_v[...] = last
        slots_v[...] = slots
        win = req
        for j in range(_NUM_REQS):
            win = jnp.where(slots == slots[j], j, win)
        last_win = plsc.load_gather(last_v, [win])
        idx_src_v[...] = last_win
        pltpu.async_copy(hs_hbm.at[idx_src_v], g3.reshape(_LANES, 2 * _KV_DIM),
                         sem).wait()
        sp = plsc.load_gather(slots_v, [lax.shift_right_logical(lanes, 1)])
        idx_dst_v[...] = sp + num_slots * lax.bitwise_and(lanes, 1)
        pltpu.async_copy(g3.at[pl.ds(0, _LANES)], cache_ref.at[idx_dst_v],
                         sem).wait()


def _make_scatter(total_tokens):
    mesh = plsc.VectorSubcoreMesh(core_axis_name="c", subcore_axis_name="s",
                                  num_cores=1)
    return pl.kernel(
        _scatter_body,
        out_type=(),
        mesh=mesh,
        compiler_params=pltpu.CompilerParams(
            needs_layout_passes=False,
            use_tc_tiling_on_sc=True,
        ),
        scratch_types=[
            pltpu.VMEM((_LANES,), jnp.int32),
            pltpu.VMEM((total_tokens,), jnp.int32),
            pltpu.VMEM((_LANES,), jnp.int32),
            pltpu.VMEM((_LANES,), jnp.int32),
            pltpu.VMEM((_LANES,), jnp.int32),
            pltpu.VMEM((_LANES,), jnp.int32),
            pltpu.VMEM((2 * _LANES, _NUM_KV_HEADS, _HEAD_SIZE), jnp.float32),
            pltpu.SemaphoreType.DMA,
        ],
    )


def _copy_block(in_ref, out_ref):
    out_ref[...] = in_ref[...]


_TC_BLK = 2048


def _tc_copy(cache3):
    total_rows = cache3.shape[0]
    return pl.pallas_call(
        _copy_block,
        grid=(total_rows // _TC_BLK,),
        in_specs=[pl.BlockSpec((_TC_BLK, _NUM_KV_HEADS, _HEAD_SIZE),
                               lambda i: (i, 0, 0))],
        out_specs=pl.BlockSpec((_TC_BLK, _NUM_KV_HEADS, _HEAD_SIZE),
                               lambda i: (i, 0, 0)),
        out_shape=jax.ShapeDtypeStruct(cache3.shape, cache3.dtype),
        compiler_params=pltpu.CompilerParams(
            dimension_semantics=("arbitrary",),
        ),
    )(cache3)


def kernel(hidden_states, kv_cache, query_start_loc, slot_mapping):
    total_tokens = hidden_states.shape[0]
    num_slots = kv_cache.shape[1]
    qslp = jnp.pad(query_start_loc, (0, _LANES - query_start_loc.shape[0]))
    cache3 = kv_cache.reshape(2 * num_slots, _NUM_KV_HEADS, _HEAD_SIZE)
    cache_ref = jax.new_ref(_tc_copy(cache3))
    _make_scatter(total_tokens)(hidden_states, qslp, slot_mapping, cache_ref)
    new_cache = jax.freeze(cache_ref).reshape(kv_cache.shape)
    output = jnp.zeros((_NUM_REQS, _NUM_HEADS * _HEAD_SIZE),
                       dtype=hidden_states.dtype)
    return new_cache, output


# speedup vs baseline: 1.0954x; 1.0011x over previous
import jax
import jax.numpy as jnp
from jax import lax
from jax.experimental import pallas as pl
from jax.experimental.pallas import tpu as pltpu
from jax.experimental.pallas import tpu_sc as plsc

_NUM_HEADS = 32
_HEAD_SIZE = 128
_NUM_KV_HEADS = 8
_NUM_REQS = 8
_KV_DIM = _NUM_KV_HEADS * _HEAD_SIZE
_LANES = 16


def _scatter_body(hs_hbm, qsl_hbm, sm_hbm, cache_ref,
                  qsl_v, sm_v, last_v, slots_v, idx_src_v, idx_dst_v,
                  g3, sem):
    num_slots = cache_ref.shape[0] // 2
    c = lax.axis_index("c")
    s = lax.axis_index("s")

    @pl.when(jnp.logical_and(c == 0, s == 0))
    def _():
        pltpu.sync_copy(qsl_hbm, qsl_v)
        pltpu.sync_copy(sm_hbm, sm_v)
        lanes = lax.iota(jnp.int32, _LANES)
        req = lax.bitwise_and(lanes, _NUM_REQS - 1)
        ends = plsc.load_gather(qsl_v, [req + 1])
        last = ends - 1
        slots = plsc.load_gather(sm_v, [last])
        last_v[...] = last
        slots_v[...] = slots
        win = req
        for j in range(_NUM_REQS):
            win = jnp.where(slots == slots[j], j, win)
        last_win = plsc.load_gather(last_v, [win])
        idx_src_v[...] = last_win
        pltpu.async_copy(hs_hbm.at[idx_src_v], g3.reshape(_LANES, 2 * _KV_DIM),
                         sem).wait()
        sp = plsc.load_gather(slots_v, [lax.shift_right_logical(lanes, 1)])
        idx_dst_v[...] = sp + num_slots * lax.bitwise_and(lanes, 1)
        pltpu.async_copy(g3.at[pl.ds(0, _LANES)], cache_ref.at[idx_dst_v],
                         sem).wait()


def _make_scatter(total_tokens):
    mesh = plsc.VectorSubcoreMesh(core_axis_name="c", subcore_axis_name="s",
                                  num_cores=1)
    return pl.kernel(
        _scatter_body,
        out_type=(),
        mesh=mesh,
        compiler_params=pltpu.CompilerParams(
            needs_layout_passes=False,
            use_tc_tiling_on_sc=True,
            skip_device_barrier=True,
            disable_bounds_checks=True,
            disable_semaphore_checks=True,
        ),
        scratch_types=[
            pltpu.VMEM((_LANES,), jnp.int32),
            pltpu.VMEM((total_tokens,), jnp.int32),
            pltpu.VMEM((_LANES,), jnp.int32),
            pltpu.VMEM((_LANES,), jnp.int32),
            pltpu.VMEM((_LANES,), jnp.int32),
            pltpu.VMEM((_LANES,), jnp.int32),
            pltpu.VMEM((2 * _LANES, _NUM_KV_HEADS, _HEAD_SIZE), jnp.float32),
            pltpu.SemaphoreType.DMA,
        ],
    )


def _copy_block(in_ref, out_ref):
    out_ref[...] = in_ref[...]


_TC_BLK = 2048


def _tc_copy(cache3):
    total_rows = cache3.shape[0]
    return pl.pallas_call(
        _copy_block,
        grid=(total_rows // _TC_BLK,),
        in_specs=[pl.BlockSpec((_TC_BLK, _NUM_KV_HEADS, _HEAD_SIZE),
                               lambda i: (i, 0, 0))],
        out_specs=pl.BlockSpec((_TC_BLK, _NUM_KV_HEADS, _HEAD_SIZE),
                               lambda i: (i, 0, 0)),
        out_shape=jax.ShapeDtypeStruct(cache3.shape, cache3.dtype),
        compiler_params=pltpu.CompilerParams(
            dimension_semantics=("arbitrary",),
        ),
    )(cache3)


def kernel(hidden_states, kv_cache, query_start_loc, slot_mapping):
    total_tokens = hidden_states.shape[0]
    num_slots = kv_cache.shape[1]
    qslp = jnp.pad(query_start_loc, (0, _LANES - query_start_loc.shape[0]))
    cache3 = kv_cache.reshape(2 * num_slots, _NUM_KV_HEADS, _HEAD_SIZE)
    cache_ref = jax.new_ref(_tc_copy(cache3))
    _make_scatter(total_tokens)(hidden_states, qslp, slot_mapping, cache_ref)
    new_cache = jax.freeze(cache_ref).reshape(kv_cache.shape)
    output = jnp.zeros((_NUM_REQS, _NUM_HEADS * _HEAD_SIZE),
                       dtype=hidden_states.dtype)
    return new_cache, output
